# baseline (speedup 1.0000x reference)
import jax
import jax.numpy as jnp
from jax import lax
from jax.experimental import pallas as pl
from jax.experimental.pallas import tpu as pltpu

_ROWS_PER_STEP = 512


def _make_tc_gather(batch, dim, rows_per_step=_ROWS_PER_STEP):
    steps = batch // rows_per_step

    def body(idx_ref, table_ref, out_ref, sem):
        i = pl.program_id(0)
        base = i * rows_per_step

        def issue(k, carry):
            r = idx_ref[base + k]
            pltpu.make_async_copy(
                table_ref.at[pl.ds(r, 1)],
                out_ref.at[pl.ds(k, 1)],
                sem,
            ).start()
            return carry

        lax.fori_loop(0, rows_per_step, issue, 0, unroll=8)
        pltpu.make_async_copy(
            table_ref.at[pl.ds(0, rows_per_step)], out_ref, sem
        ).wait()

    grid_spec = pltpu.PrefetchScalarGridSpec(
        num_scalar_prefetch=1,
        grid=(steps,),
        in_specs=[pl.BlockSpec(memory_space=pl.ANY)],
        out_specs=pl.BlockSpec((rows_per_step, dim), lambda i, idx_ref: (i, 0)),
        scratch_shapes=[pltpu.SemaphoreType.DMA],
    )
    return pl.pallas_call(
        body,
        grid_spec=grid_spec,
        out_shape=jax.ShapeDtypeStruct((batch, dim), jnp.float32),
    )


def kernel(input_ids, entity, entity_table, relation_table):
    ids = input_ids.astype(jnp.int32)
    batch = ids.shape[0]
    dim = entity_table.shape[1]

    gather = _make_tc_gather(batch, dim)
    n_rel = relation_table.shape[0]
    return lax.cond(
        entity != 0,
        lambda: gather(ids, entity_table),
        lambda: gather(jnp.clip(ids, 0, n_rel - 1), relation_table),
    )


# speedup vs baseline: 1.1801x; 1.1801x over previous
import functools

import jax
import jax.numpy as jnp
from jax import lax
from jax.experimental import pallas as pl
from jax.experimental.pallas import tpu as pltpu
from jax.experimental.pallas import tpu_sc as plsc

_UNROLL = 32


def _make_sc_gather(batch, dim):
    info = plsc.get_sparse_core_info()
    nc, ns = info.num_cores, info.num_subcores
    nw = nc * ns
    b_w = batch // nw
    assert batch % (nw * _UNROLL) == 0

    mesh = plsc.VectorSubcoreMesh(core_axis_name="c", subcore_axis_name="s")

    @functools.partial(
        pl.kernel,
        mesh=mesh,
        out_type=jax.ShapeDtypeStruct((batch, dim), jnp.float32),
        scratch_types=[
            pltpu.VMEM((b_w,), jnp.int32),
            pltpu.VMEM((b_w, dim), jnp.float32),
            pltpu.SemaphoreType.DMA,
            pltpu.SemaphoreType.DMA,
        ],
    )
    def gather(table_hbm, idx_hbm, out_hbm, idx_v, rows_v, sem, sem2):
        wid = lax.axis_index("s") * nc + lax.axis_index("c")
        base = wid * b_w
        pltpu.sync_copy(idx_hbm.at[pl.ds(base, b_w)], idx_v)

        def body(j, carry):
            vec = idx_v[pl.ds(j * _UNROLL, _UNROLL)]
            for t in range(_UNROLL):
                pltpu.async_copy(
                    table_hbm.at[pl.ds(vec[t], 1)],
                    rows_v.at[pl.ds(j * _UNROLL + t, 1)],
                    sem if t % 2 == 0 else sem2,
                )
            return carry

        lax.fori_loop(0, b_w // _UNROLL, body, 0)
        half = b_w // 2
        pltpu.make_async_copy(
            table_hbm.at[pl.ds(0, half)], rows_v.at[pl.ds(0, half)], sem
        ).wait()
        pltpu.make_async_copy(
            table_hbm.at[pl.ds(0, half)], rows_v.at[pl.ds(0, half)], sem2
        ).wait()
        pltpu.sync_copy(rows_v, out_hbm.at[pl.ds(base, b_w)])

    return gather


def kernel(input_ids, entity, entity_table, relation_table):
    ids = input_ids.astype(jnp.int32)
    batch = ids.shape[0]
    dim = entity_table.shape[1]

    gather = _make_sc_gather(batch, dim)
    n_rel = relation_table.shape[0]
    return lax.cond(
        entity != 0,
        lambda: gather(entity_table, ids),
        lambda: gather(relation_table, jnp.clip(ids, 0, n_rel - 1)),
    )
